# baseline (speedup 1.0000x reference)
import functools

import jax
import jax.numpy as jnp
from jax import lax
from jax.experimental import pallas as pl
from jax.experimental.pallas import tpu as pltpu
from jax.experimental.pallas import tpu_sc as plsc

N = 50000
E = 800000
H = 4
C = 16
EPS = 1e-5

NC = 2
NS = 16
PKW = 48
ALDW = 16
K = 112
ET = E // NS
NCHUNK = 447
PT = NCHUNK * K
NNUM = N + 8
NDEN = 6256
WB = 80
NWB = N // WB
DW = 50
NWD = (N // 8) // DW


def _edge_body(pack0, pack1, ald0, ald1, eidx_h,
               accn0, accn1, accd0, accd1,
               num_sh, den_sh, idx0, idx1, dsts0, dsts1, dst40, dst41,
               rows0, rows1, ald_0, ald_1, msgv, denv,
               sidx0, sidx1, srow, sald, ssc):
    c = lax.axis_index("c")
    s = lax.axis_index("s")
    zeros16 = jnp.zeros((16,), jnp.float32)
    iota16 = lax.iota(jnp.int32, 16)
    idxb = (idx0, idx1)
    dstsb = (dsts0, dsts1)
    dst4b = (dst40, dst41)
    rowsb = (rows0, rows1)
    aldb = (ald_0, ald_1)

    def zrow(i, _):
        msgv[i, pl.ds(0, 16)] = zeros16
        msgv[i, pl.ds(16, 16)] = zeros16
        denv[i, pl.ds(0, 16)] = zeros16
        return 0
    lax.fori_loop(0, K, zrow, 0)

    def zacc(j, _):
        @pl.when(j % NS == s)
        def _():
            pltpu.sync_copy(msgv.at[pl.ds(0, WB)], num_sh.at[pl.ds(j * WB, WB)])
        return 0
    lax.fori_loop(0, NWB, zacc, 0)

    def zaccd(j, _):
        @pl.when(j % NS == s)
        def _():
            pltpu.sync_copy(denv.at[pl.ds(0, DW)], den_sh.at[pl.ds(j * DW, DW)])
        return 0
    lax.fori_loop(0, NWD, zaccd, 0)

    @pl.when(s == 0)
    def _():
        pltpu.sync_copy(msgv.at[pl.ds(0, 8)], num_sh.at[pl.ds(N, 8)])
        pltpu.sync_copy(denv.at[pl.ds(0, 16)], den_sh.at[pl.ds(NDEN - 16, 16)])
    plsc.subcore_barrier()

    sidxb = (sidx0, sidx1)

    def idx_slice(k):
        return eidx_h.at[:, pl.ds(s * PT + k * K, K)]

    def start_idx(k, p):
        pltpu.async_copy(idx_slice(k), idxb[p], sidxb[p])

    def wait_idx(k, p):
        pltpu.make_async_copy(idx_slice(k), idxb[p], sidxb[p]).wait()

    GSPLIT = ((0, 32), (32, 32), (64, 32), (96, 16))
    GA = K // 2

    def start_gather(p):
        @pl.when(c == 0)
        def _():
            for o, n in GSPLIT:
                pltpu.async_copy(pack0.at[idxb[p].at[0, pl.ds(o, n)]],
                                 rowsb[p].at[pl.ds(o, n)], srow)
            for q in range(2):
                pltpu.async_copy(ald0.at[idxb[p].at[1, pl.ds(q * GA, GA)]],
                                 aldb[p].at[pl.ds(q * GA, GA)], sald)

        @pl.when(c == 1)
        def _():
            for o, n in GSPLIT:
                pltpu.async_copy(pack1.at[idxb[p].at[0, pl.ds(o, n)]],
                                 rowsb[p].at[pl.ds(o, n)], srow)
            for q in range(2):
                pltpu.async_copy(ald1.at[idxb[p].at[1, pl.ds(q * GA, GA)]],
                                 aldb[p].at[pl.ds(q * GA, GA)], sald)

    def wait_gather(p):
        for o, n in GSPLIT:
            pltpu.make_async_copy(pack0.at[idxb[p].at[0, pl.ds(o, n)]],
                                  rowsb[p].at[pl.ds(o, n)], srow).wait()
        for q in range(2):
            pltpu.make_async_copy(ald0.at[idxb[p].at[1, pl.ds(q * GA, GA)]],
                                  aldb[p].at[pl.ds(q * GA, GA)], sald).wait()

    def start_scatter(p):
        pltpu.async_copy(msgv, num_sh.at[dstsb[p]], ssc, add=True)
        pltpu.async_copy(denv, den_sh.at[dst4b[p]], ssc, add=True)

    def wait_scatter(p):
        pltpu.make_async_copy(msgv, num_sh.at[dstsb[p]], ssc).wait()
        pltpu.make_async_copy(denv, den_sh.at[dst4b[p]], ssc).wait()

    def copy_dst(p):
        def cp(g, _):
            dstm = idxb[p][1, pl.ds(g * 16, 16)]
            dstsb[p][pl.ds(g * 16, 16)] = dstm
            dst4b[p][pl.ds(g * 16, 16)] = lax.shift_right_logical(dstm, 3)
            return 0
        lax.fori_loop(0, K // 16, cp, 0)

    def compute(p):
        def grp(g, _):
            e16 = g * 16 + iota16
            dstm = dstsb[p][pl.ds(g * 16, 16)]
            colbase = lax.shift_left(
                lax.bitwise_and(dstm, jnp.full((16,), 7, jnp.int32)), 1)
            for hh in range(2):
                als = plsc.load_gather(
                    rowsb[p], [e16, jnp.full((16,), 32 + hh, jnp.int32)])
                ad = plsc.load_gather(
                    aldb[p], [e16, jnp.full((16,), hh, jnp.int32)])
                e = als + ad
                e = jnp.where(e > 0, e, 0.2 * e)
                w = jnp.exp(e)
                plsc.store_scatter(denv, [e16, colbase + hh], w)
                for colj in range(16):
                    cc = jnp.full((16,), hh * 16 + colj, jnp.int32)
                    hv = plsc.load_gather(rowsb[p], [e16, cc])
                    plsc.store_scatter(msgv, [e16, cc], w * hv)
            return 0
        lax.fori_loop(0, K // 16, grp, 0)

    def rezero_den():
        def zden(i, _):
            denv[i, pl.ds(0, 16)] = zeros16
            return 0
        lax.fori_loop(0, K, zden, 0)

    start_idx(0, 0)
    start_idx(1, 1)
    wait_idx(0, 0)
    start_gather(0)

    def pipe(k, p, first):
        wait_gather(p)
        wait_idx(k + 1, 1 - p)
        start_gather(1 - p)
        copy_dst(p)

        @pl.when(k + 2 < NCHUNK)
        def _():
            start_idx(k + 2, p)
        if not first:
            wait_scatter(1 - p)
            rezero_den()
        compute(p)
        start_scatter(p)

    pipe(0, 0, True)
    pipe(1, 1, False)

    def dbl(k2, _):
        k = 2 * k2
        pipe(k, 0, False)
        pipe(k + 1, 1, False)
        return 0
    lax.fori_loop(1, (NCHUNK - 1) // 2, dbl, 0)

    wait_gather(0)
    copy_dst(0)
    wait_scatter(1)
    rezero_den()
    compute(0)
    start_scatter(0)
    wait_scatter(0)
    plsc.subcore_barrier()

    def wb(j, _):
        @pl.when(j % NS == s)
        def _():
            r0 = j * WB
            pltpu.sync_copy(num_sh.at[pl.ds(r0, WB)], msgv.at[pl.ds(0, WB)])

            @pl.when(c == 0)
            def _():
                pltpu.sync_copy(msgv.at[pl.ds(0, WB)], accn0.at[pl.ds(r0, WB)])

            @pl.when(c == 1)
            def _():
                pltpu.sync_copy(msgv.at[pl.ds(0, WB)], accn1.at[pl.ds(r0, WB)])
        return 0
    lax.fori_loop(0, NWB, wb, 0)

    def wbd(j, _):
        @pl.when(j % NS == s)
        def _():
            r0 = j * DW
            pltpu.sync_copy(den_sh.at[pl.ds(r0, DW)], denv.at[pl.ds(0, DW)])

            @pl.when(c == 0)
            def _():
                pltpu.sync_copy(denv.at[pl.ds(0, DW)], accd0.at[pl.ds(r0, DW)])

            @pl.when(c == 1)
            def _():
                pltpu.sync_copy(denv.at[pl.ds(0, DW)], accd1.at[pl.ds(r0, DW)])
        return 0
    lax.fori_loop(0, NWD, wbd, 0)


@jax.jit
def _edge_pass(pack0, pack1, ald0, ald1, eidx):
    mesh = plsc.VectorSubcoreMesh(core_axis_name="c", subcore_axis_name="s")
    f = pl.kernel(
        _edge_body,
        out_type=(jax.ShapeDtypeStruct((N, 32), jnp.float32),
                  jax.ShapeDtypeStruct((N, 32), jnp.float32),
                  jax.ShapeDtypeStruct((N // 8, 16), jnp.float32),
                  jax.ShapeDtypeStruct((N // 8, 16), jnp.float32)),
        mesh=mesh,
        compiler_params=pltpu.CompilerParams(
            needs_layout_passes=False, use_tc_tiling_on_sc=False),
        scratch_types=[
            pltpu.VMEM_SHARED((NNUM, 32), jnp.float32),
            pltpu.VMEM_SHARED((NDEN, 16), jnp.float32),
            pltpu.VMEM((2, K), jnp.int32),
            pltpu.VMEM((2, K), jnp.int32),
            pltpu.VMEM((K,), jnp.int32),
            pltpu.VMEM((K,), jnp.int32),
            pltpu.VMEM((K,), jnp.int32),
            pltpu.VMEM((K,), jnp.int32),
            pltpu.VMEM((K, PKW), jnp.float32),
            pltpu.VMEM((K, PKW), jnp.float32),
            pltpu.VMEM((K, ALDW), jnp.float32),
            pltpu.VMEM((K, ALDW), jnp.float32),
            pltpu.VMEM((K, 32), jnp.float32),
            pltpu.VMEM((K, 16), jnp.float32),
            pltpu.SemaphoreType.DMA,
            pltpu.SemaphoreType.DMA,
            pltpu.SemaphoreType.DMA,
            pltpu.SemaphoreType.DMA,
            pltpu.SemaphoreType.DMA,
        ],
    )
    return f(pack0, pack1, ald0, ald1, eidx)


def _pack_mats(Wg, a_s, a_d):
    HC = H * C
    Ms, Ds = [], []
    for c in range(NC):
        P = jnp.zeros((HC, PKW), jnp.float32)
        P = P.at[c * 32:(c + 1) * 32, 0:32].set(jnp.eye(32))
        for hh in range(2):
            head = c * 2 + hh
            P = P.at[head * C:(head + 1) * C, 32 + hh].set(a_s[head])
        D = jnp.zeros((HC, ALDW), jnp.float32)
        for hh in range(2):
            head = c * 2 + hh
            D = D.at[head * C:(head + 1) * C, hh].set(a_d[head])
        Ms.append(Wg @ P)
        Ds.append(Wg @ D)
    return Ms, Ds


def _gat_sc(x, eidx, Wg, a_s, a_d, b, concat):
    Ms, Ds = _pack_mats(Wg, a_s, a_d)
    accn0, accn1, accd0, accd1 = _edge_pass(
        x @ Ms[0], x @ Ms[1], x @ Ds[0], x @ Ds[1], eidx)
    num = jnp.concatenate([accn0, accn1], axis=1)
    den = jnp.concatenate([accd0.reshape(N, 2),
                           accd1.reshape(N, 2)], axis=1)
    out = num.reshape(N, H, C) / (den[:, :, None] + 1e-16)
    if concat:
        out = out.reshape(N, H * C)
    else:
        out = out.mean(axis=1)
    return out + b


def _bn(x, g, b):
    m = x.mean(0)
    v = x.var(0)
    return (x - m) / jnp.sqrt(v + EPS) * g + b


def kernel(X, edge_index, W_in, b_in, Wg0, as0, ad0, bg0, g0, be0,
           Wg1, as1, ad1, bg1, g1, be1, Wo1, bo1, Wo2, bo2):
    pad = jnp.broadcast_to(
        jnp.array([0, N], jnp.int32).reshape(2, 1, 1), (2, NS, PT - ET))
    eidx = jnp.concatenate(
        [edge_index.reshape(2, NS, ET), pad], axis=2).reshape(2, NS * PT)
    x = X @ W_in + b_in
    x = _gat_sc(x, eidx, Wg0, as0, ad0, bg0, True)
    x = _bn(x, g0, be0)
    x = jax.nn.relu(x)
    x = _gat_sc(x, eidx, Wg1, as1, ad1, bg1, False)
    x = _bn(x, g1, be1)
    h = jax.nn.relu(x @ Wo1 + bo1)
    out = h @ Wo2 + bo2
    return out


# speedup vs baseline: 1.4132x; 1.4132x over previous
import functools

import jax
import jax.numpy as jnp
from jax import lax
from jax.experimental import pallas as pl
from jax.experimental.pallas import tpu as pltpu
from jax.experimental.pallas import tpu_sc as plsc

N = 50000
E = 800000
H = 4
C = 16
EPS = 1e-5

NC = 2
NS = 16
PKW = 48
ALDW = 16
K = 112
ET = E // NS
NCHUNK = 447
PT = NCHUNK * K
NNUM = N + 8
NDEN = 6256
WB = 80
NWB = N // WB
DW = 50
NWD = (N // 8) // DW


def _edge_body(pack0, pack1, ald0, ald1, eidx_h,
               accn0, accn1, accd0, accd1,
               num_sh, den_sh, idx0, idx1, dsts0, dsts1, dst40, dst41,
               rows0, rows1, ald_0, ald_1, msgv, denv,
               sidx0, sidx1, srow, sald, ssc):
    c = lax.axis_index("c")
    s = lax.axis_index("s")
    zeros16 = jnp.zeros((16,), jnp.float32)
    iota16 = lax.iota(jnp.int32, 16)
    idxb = (idx0, idx1)
    dstsb = (dsts0, dsts1)
    dst4b = (dst40, dst41)
    rowsb = (rows0, rows1)
    aldb = (ald_0, ald_1)

    def zrow(i, _):
        msgv[i, pl.ds(0, 16)] = zeros16
        msgv[i, pl.ds(16, 16)] = zeros16
        denv[i, pl.ds(0, 16)] = zeros16
        return 0
    lax.fori_loop(0, K, zrow, 0)

    def zacc(j, _):
        @pl.when(j % NS == s)
        def _():
            pltpu.sync_copy(msgv.at[pl.ds(0, WB)], num_sh.at[pl.ds(j * WB, WB)])
        return 0
    lax.fori_loop(0, NWB, zacc, 0)

    def zaccd(j, _):
        @pl.when(j % NS == s)
        def _():
            pltpu.sync_copy(denv.at[pl.ds(0, DW)], den_sh.at[pl.ds(j * DW, DW)])
        return 0
    lax.fori_loop(0, NWD, zaccd, 0)

    @pl.when(s == 0)
    def _():
        pltpu.sync_copy(msgv.at[pl.ds(0, 8)], num_sh.at[pl.ds(N, 8)])
        pltpu.sync_copy(denv.at[pl.ds(0, 16)], den_sh.at[pl.ds(NDEN - 16, 16)])
    plsc.subcore_barrier()

    sidxb = (sidx0, sidx1)

    def idx_slice(k):
        return eidx_h.at[:, pl.ds(s * PT + k * K, K)]

    def start_idx(k, p):
        pltpu.async_copy(idx_slice(k), idxb[p], sidxb[p])

    def wait_idx(k, p):
        pltpu.make_async_copy(idx_slice(k), idxb[p], sidxb[p]).wait()

    GSPLIT = ((0, 32), (32, 32), (64, 32), (96, 16))
    GA = K // 2

    def start_gather(p):
        @pl.when(c == 0)
        def _():
            for o, n in GSPLIT:
                pltpu.async_copy(pack0.at[idxb[p].at[0, pl.ds(o, n)]],
                                 rowsb[p].at[pl.ds(o, n)], srow)
            for q in range(2):
                pltpu.async_copy(ald0.at[idxb[p].at[1, pl.ds(q * GA, GA)]],
                                 aldb[p].at[pl.ds(q * GA, GA)], sald)

        @pl.when(c == 1)
        def _():
            for o, n in GSPLIT:
                pltpu.async_copy(pack1.at[idxb[p].at[0, pl.ds(o, n)]],
                                 rowsb[p].at[pl.ds(o, n)], srow)
            for q in range(2):
                pltpu.async_copy(ald1.at[idxb[p].at[1, pl.ds(q * GA, GA)]],
                                 aldb[p].at[pl.ds(q * GA, GA)], sald)

    def wait_gather(p):
        for o, n in GSPLIT:
            pltpu.make_async_copy(pack0.at[idxb[p].at[0, pl.ds(o, n)]],
                                  rowsb[p].at[pl.ds(o, n)], srow).wait()
        for q in range(2):
            pltpu.make_async_copy(ald0.at[idxb[p].at[1, pl.ds(q * GA, GA)]],
                                  aldb[p].at[pl.ds(q * GA, GA)], sald).wait()

    def start_scatter(p):
        pltpu.async_copy(msgv, num_sh.at[dstsb[p]], ssc, add=True)
        pltpu.async_copy(denv, den_sh.at[dst4b[p]], ssc, add=True)

    def wait_scatter(p):
        pltpu.make_async_copy(msgv, num_sh.at[dstsb[p]], ssc).wait()
        pltpu.make_async_copy(denv, den_sh.at[dst4b[p]], ssc).wait()

    def copy_dst(p):
        def cp(g, _):
            dstm = idxb[p][1, pl.ds(g * 16, 16)]
            dstsb[p][pl.ds(g * 16, 16)] = dstm
            dst4b[p][pl.ds(g * 16, 16)] = lax.shift_right_logical(dstm, 3)
            return 0
        lax.fori_loop(0, K // 16, cp, 0)

    def compute(p):
        def grp(g, _):
            e16 = g * 16 + iota16
            dstm = dstsb[p][pl.ds(g * 16, 16)]
            colbase = lax.shift_left(
                lax.bitwise_and(dstm, jnp.full((16,), 7, jnp.int32)), 1)
            ws = []
            for hh in range(2):
                als = plsc.load_gather(
                    rowsb[p], [e16, jnp.full((16,), 32 + hh, jnp.int32)])
                ad = plsc.load_gather(
                    aldb[p], [e16, jnp.full((16,), hh, jnp.int32)])
                e = als + ad
                e = jnp.where(e > 0, e, 0.2 * e)
                ws.append(jnp.exp(e))
            for hh in range(2):
                plsc.store_scatter(denv, [e16, colbase + hh], ws[hh])
            hvs = [plsc.load_gather(rowsb[p],
                                    [e16, jnp.full((16,), col, jnp.int32)])
                   for col in range(32)]
            for col in range(32):
                cc = jnp.full((16,), col, jnp.int32)
                plsc.store_scatter(msgv, [e16, cc], ws[col // 16] * hvs[col])
            return 0
        lax.fori_loop(0, K // 16, grp, 0)

    def rezero_den():
        def zden(i, _):
            denv[i, pl.ds(0, 16)] = zeros16
            return 0
        lax.fori_loop(0, K, zden, 0)

    start_idx(0, 0)
    start_idx(1, 1)
    wait_idx(0, 0)
    start_gather(0)

    def pipe(k, p, first):
        wait_gather(p)
        wait_idx(k + 1, 1 - p)
        start_gather(1 - p)
        copy_dst(p)

        @pl.when(k + 2 < NCHUNK)
        def _():
            start_idx(k + 2, p)
        if not first:
            wait_scatter(1 - p)
            rezero_den()
        compute(p)
        start_scatter(p)

    pipe(0, 0, True)
    pipe(1, 1, False)

    def dbl(k2, _):
        k = 2 * k2
        pipe(k, 0, False)
        pipe(k + 1, 1, False)
        return 0
    lax.fori_loop(1, (NCHUNK - 1) // 2, dbl, 0)

    wait_gather(0)
    copy_dst(0)
    wait_scatter(1)
    rezero_den()
    compute(0)
    start_scatter(0)
    wait_scatter(0)
    plsc.subcore_barrier()

    def wb(j, _):
        @pl.when(j % NS == s)
        def _():
            r0 = j * WB
            pltpu.sync_copy(num_sh.at[pl.ds(r0, WB)], msgv.at[pl.ds(0, WB)])

            @pl.when(c == 0)
            def _():
                pltpu.sync_copy(msgv.at[pl.ds(0, WB)], accn0.at[pl.ds(r0, WB)])

            @pl.when(c == 1)
            def _():
                pltpu.sync_copy(msgv.at[pl.ds(0, WB)], accn1.at[pl.ds(r0, WB)])
        return 0
    lax.fori_loop(0, NWB, wb, 0)

    def wbd(j, _):
        @pl.when(j % NS == s)
        def _():
            r0 = j * DW
            pltpu.sync_copy(den_sh.at[pl.ds(r0, DW)], denv.at[pl.ds(0, DW)])

            @pl.when(c == 0)
            def _():
                pltpu.sync_copy(denv.at[pl.ds(0, DW)], accd0.at[pl.ds(r0, DW)])

            @pl.when(c == 1)
            def _():
                pltpu.sync_copy(denv.at[pl.ds(0, DW)], accd1.at[pl.ds(r0, DW)])
        return 0
    lax.fori_loop(0, NWD, wbd, 0)


@jax.jit
def _edge_pass(pack0, pack1, ald0, ald1, eidx):
    mesh = plsc.VectorSubcoreMesh(core_axis_name="c", subcore_axis_name="s")
    f = pl.kernel(
        _edge_body,
        out_type=(jax.ShapeDtypeStruct((N, 32), jnp.float32),
                  jax.ShapeDtypeStruct((N, 32), jnp.float32),
                  jax.ShapeDtypeStruct((N // 8, 16), jnp.float32),
                  jax.ShapeDtypeStruct((N // 8, 16), jnp.float32)),
        mesh=mesh,
        compiler_params=pltpu.CompilerParams(
            needs_layout_passes=False, use_tc_tiling_on_sc=False),
        scratch_types=[
            pltpu.VMEM_SHARED((NNUM, 32), jnp.float32),
            pltpu.VMEM_SHARED((NDEN, 16), jnp.float32),
            pltpu.VMEM((2, K), jnp.int32),
            pltpu.VMEM((2, K), jnp.int32),
            pltpu.VMEM((K,), jnp.int32),
            pltpu.VMEM((K,), jnp.int32),
            pltpu.VMEM((K,), jnp.int32),
            pltpu.VMEM((K,), jnp.int32),
            pltpu.VMEM((K, PKW), jnp.float32),
            pltpu.VMEM((K, PKW), jnp.float32),
            pltpu.VMEM((K, ALDW), jnp.float32),
            pltpu.VMEM((K, ALDW), jnp.float32),
            pltpu.VMEM((K, 32), jnp.float32),
            pltpu.VMEM((K, 16), jnp.float32),
            pltpu.SemaphoreType.DMA,
            pltpu.SemaphoreType.DMA,
            pltpu.SemaphoreType.DMA,
            pltpu.SemaphoreType.DMA,
            pltpu.SemaphoreType.DMA,
        ],
    )
    return f(pack0, pack1, ald0, ald1, eidx)


def _pack_mats(Wg, a_s, a_d):
    HC = H * C
    Ms, Ds = [], []
    for c in range(NC):
        P = jnp.zeros((HC, PKW), jnp.float32)
        P = P.at[c * 32:(c + 1) * 32, 0:32].set(jnp.eye(32))
        for hh in range(2):
            head = c * 2 + hh
            P = P.at[head * C:(head + 1) * C, 32 + hh].set(a_s[head])
        D = jnp.zeros((HC, ALDW), jnp.float32)
        for hh in range(2):
            head = c * 2 + hh
            D = D.at[head * C:(head + 1) * C, hh].set(a_d[head])
        Ms.append(Wg @ P)
        Ds.append(Wg @ D)
    return Ms, Ds


def _gat_sc(x, eidx, Wg, a_s, a_d, b, concat):
    Ms, Ds = _pack_mats(Wg, a_s, a_d)
    accn0, accn1, accd0, accd1 = _edge_pass(
        x @ Ms[0], x @ Ms[1], x @ Ds[0], x @ Ds[1], eidx)
    num = jnp.concatenate([accn0, accn1], axis=1)
    den = jnp.concatenate([accd0.reshape(N, 2),
                           accd1.reshape(N, 2)], axis=1)
    out = num.reshape(N, H, C) / (den[:, :, None] + 1e-16)
    if concat:
        out = out.reshape(N, H * C)
    else:
        out = out.mean(axis=1)
    return out + b


def _bn(x, g, b):
    m = x.mean(0)
    v = x.var(0)
    return (x - m) / jnp.sqrt(v + EPS) * g + b


def kernel(X, edge_index, W_in, b_in, Wg0, as0, ad0, bg0, g0, be0,
           Wg1, as1, ad1, bg1, g1, be1, Wo1, bo1, Wo2, bo2):
    pad = jnp.broadcast_to(
        jnp.array([0, N], jnp.int32).reshape(2, 1, 1), (2, NS, PT - ET))
    eidx = jnp.concatenate(
        [edge_index.reshape(2, NS, ET), pad], axis=2).reshape(2, NS * PT)
    x = X @ W_in + b_in
    x = _gat_sc(x, eidx, Wg0, as0, ad0, bg0, True)
    x = _bn(x, g0, be0)
    x = jax.nn.relu(x)
    x = _gat_sc(x, eidx, Wg1, as1, ad1, bg1, False)
    x = _bn(x, g1, be1)
    h = jax.nn.relu(x @ Wo1 + bo1)
    out = h @ Wo2 + bo2
    return out
